# baseline (speedup 1.0000x reference)
import functools

import jax
import jax.numpy as jnp
from jax.experimental import pallas as pl
from jax.experimental.pallas import tpu as pltpu

B, S, D, NREC = 4, 8192, 768, 2
N = B * S


def _fused_kernel(x_ref, wr_ref, wk_ref, out_ref, stats_ref):
    i = pl.program_id(0)
    x = x_ref[...]

    logits = jax.lax.dot_general(
        x.astype(jnp.bfloat16), wr_ref[...].astype(jnp.bfloat16),
        (((1,), (1,)), ((), ())),
        preferred_element_type=jnp.float32,
    )
    l0 = logits[:, 0:1]
    l1 = logits[:, 1:2]

    m = jnp.maximum(l0, l1)
    ea = jnp.exp(-jnp.abs(l1 - l0))
    denom = 1.0 + ea
    w = 1.0 / denom
    lse = m + jnp.log(denom)
    take1 = l1 > l0

    t1f = take1.astype(jnp.float32)
    g1 = w * t1f
    g0 = w - g1
    xg = jnp.concatenate([x * g0, x * g1], axis=1).astype(jnp.bfloat16)
    proc = jax.lax.dot_general(
        xg, wk_ref[...],
        (((1,), (0,)), ((), ())),
        preferred_element_type=jnp.float32,
    )
    out_ref[...] = x + proc

    pmin = ea * w
    p1 = jnp.where(take1, w, pmin)
    p0 = jnp.where(take1, pmin, w)
    part = jnp.stack([
        jnp.sum(lse * lse),
        jnp.sum(p0),
        jnp.sum(p1),
        jnp.sum(t1f),
        jnp.zeros((), jnp.float32),
        jnp.zeros((), jnp.float32),
        jnp.zeros((), jnp.float32),
        jnp.zeros((), jnp.float32),
    ])[:, None] * jnp.ones((8, 128), jnp.float32)

    @pl.when(i == 0)
    def _init():
        stats_ref[...] = part

    @pl.when(i > 0)
    def _acc():
        stats_ref[...] += part


@functools.partial(jax.jit, static_argnames=("block_t",))
def _run(hidden_states, Wr, Wb0, Wb1, block_t=1024):
    flat = hidden_states.reshape(N, D)
    wk = jnp.concatenate([Wb0, Wb1], axis=0).astype(jnp.bfloat16)
    grid = N // block_t
    out, stats = pl.pallas_call(
        _fused_kernel,
        grid=(grid,),
        in_specs=[
            pl.BlockSpec((block_t, D), lambda i: (i, 0)),
            pl.BlockSpec((NREC, D), lambda i: (0, 0)),
            pl.BlockSpec((2 * D, D), lambda i: (0, 0)),
        ],
        out_specs=[
            pl.BlockSpec((block_t, D), lambda i: (i, 0)),
            pl.BlockSpec((8, 128), lambda i: (0, 0)),
        ],
        out_shape=[
            jax.ShapeDtypeStruct((N, D), jnp.float32),
            jax.ShapeDtypeStruct((8, 128), jnp.float32),
        ],
        compiler_params=pltpu.CompilerParams(
            dimension_semantics=("arbitrary",),
        ),
    )(flat, Wr, wk)

    lse2_sum = stats[0, 0]
    p0_sum = stats[1, 0]
    p1_sum = stats[2, 0]
    cnt1 = stats[3, 0]
    cnt0 = jnp.float32(N) - cnt1

    router_z_loss = lse2_sum / N
    expert_probs = jnp.stack([p0_sum, p1_sum]) / N
    expert_freq = jnp.stack([cnt0, cnt1]) / N
    balancing_loss = jnp.sum(expert_probs * expert_freq) * 0.1

    return out.reshape(B, S, D), router_z_loss, balancing_loss


def kernel(hidden_states, Wr, Wb0, Wb1):
    return _run(hidden_states, Wr, Wb0, Wb1)


# speedup vs baseline: 1.1136x; 1.1136x over previous
import functools

import jax
import jax.numpy as jnp
from jax.experimental import pallas as pl
from jax.experimental.pallas import tpu as pltpu

B, S, D, NREC = 4, 8192, 768, 2
N = B * S


def _fused_kernel(x_ref, wr_ref, wcat_ref, out_ref, stats_ref):
    i = pl.program_id(0)
    x = x_ref[...]

    logits = jax.lax.dot_general(
        x.astype(jnp.bfloat16), wr_ref[...].astype(jnp.bfloat16),
        (((1,), (1,)), ((), ())),
        preferred_element_type=jnp.float32,
    )
    l0 = logits[:, 0]
    l1 = logits[:, 1]
    m = jnp.maximum(l0, l1)
    ea = jnp.exp(-jnp.abs(l1 - l0))
    denom = 1.0 + ea
    w = 1.0 / denom
    lse = m + jnp.log(denom)
    take1 = l1 > l0

    proc = jax.lax.dot_general(
        x.astype(jnp.bfloat16), wcat_ref[...],
        (((1,), (0,)), ((), ())),
        preferred_element_type=jnp.float32,
    )
    sel = jnp.where(take1[:, None], proc[:, D:], proc[:, :D])
    out_ref[...] = x + w[:, None] * sel

    pmin = ea * w
    t1f = take1.astype(jnp.float32)
    p1 = jnp.where(take1, w, pmin)
    p0 = jnp.where(take1, pmin, w)
    part = jnp.stack([
        jnp.sum(lse * lse),
        jnp.sum(p0),
        jnp.sum(p1),
        jnp.sum(t1f),
        jnp.zeros((), jnp.float32),
        jnp.zeros((), jnp.float32),
        jnp.zeros((), jnp.float32),
        jnp.zeros((), jnp.float32),
    ])[:, None] * jnp.ones((8, 128), jnp.float32)

    @pl.when(i == 0)
    def _init():
        stats_ref[...] = part

    @pl.when(i > 0)
    def _acc():
        stats_ref[...] += part


@functools.partial(jax.jit, static_argnames=("block_t",))
def _run(hidden_states, Wr, Wb0, Wb1, block_t=512):
    flat = hidden_states.reshape(N, D)
    wcat = jnp.concatenate([Wb0, Wb1], axis=1).astype(jnp.bfloat16)
    grid = N // block_t
    out, stats = pl.pallas_call(
        _fused_kernel,
        grid=(grid,),
        in_specs=[
            pl.BlockSpec((block_t, D), lambda i: (i, 0)),
            pl.BlockSpec((NREC, D), lambda i: (0, 0)),
            pl.BlockSpec((D, 2 * D), lambda i: (0, 0)),
        ],
        out_specs=[
            pl.BlockSpec((block_t, D), lambda i: (i, 0)),
            pl.BlockSpec((8, 128), lambda i: (0, 0)),
        ],
        out_shape=[
            jax.ShapeDtypeStruct((N, D), jnp.float32),
            jax.ShapeDtypeStruct((8, 128), jnp.float32),
        ],
        compiler_params=pltpu.CompilerParams(
            dimension_semantics=("arbitrary",),
        ),
    )(flat, Wr, wcat)

    lse2_sum = stats[0, 0]
    p0_sum = stats[1, 0]
    p1_sum = stats[2, 0]
    cnt1 = stats[3, 0]
    cnt0 = jnp.float32(N) - cnt1

    router_z_loss = lse2_sum / N
    expert_probs = jnp.stack([p0_sum, p1_sum]) / N
    expert_freq = jnp.stack([cnt0, cnt1]) / N
    balancing_loss = jnp.sum(expert_probs * expert_freq) * 0.1

    return out.reshape(B, S, D), router_z_loss, balancing_loss


def kernel(hidden_states, Wr, Wb0, Wb1):
    return _run(hidden_states, Wr, Wb0, Wb1)


# speedup vs baseline: 1.2169x; 1.0928x over previous
import functools

import jax
import jax.numpy as jnp
from jax.experimental import pallas as pl
from jax.experimental.pallas import tpu as pltpu

B, S, D, NREC = 4, 8192, 768, 2
N = B * S


def _fused_kernel(x_ref, wr_ref, wcat_ref, out_ref, stats_ref):
    i = pl.program_id(0)
    x = x_ref[...]

    logits = jax.lax.dot_general(
        x.astype(jnp.bfloat16), wr_ref[...].astype(jnp.bfloat16),
        (((1,), (1,)), ((), ())),
        preferred_element_type=jnp.float32,
    )
    l0 = logits[:, 0:1]
    l1 = logits[:, 1:2]
    d = l1 - l0
    m = jnp.maximum(l0, l1)
    ea = jnp.exp(-jnp.abs(d))
    denom = 1.0 + ea
    w = 1.0 / denom
    lse = m + jnp.log(denom)
    take1 = d > 0.0

    xb = x.astype(jnp.bfloat16)
    for j in range(D // 128):
        proc_j = jax.lax.dot_general(
            xb, wcat_ref[:, 256 * j:256 * (j + 1)],
            (((1,), (0,)), ((), ())),
            preferred_element_type=jnp.float32,
        )
        sel_j = jnp.where(take1, proc_j[:, 128:], proc_j[:, :128])
        out_ref[:, 128 * j:128 * (j + 1)] = (
            x[:, 128 * j:128 * (j + 1)] + w * sel_j)

    pmin = ea * w
    t1f = take1.astype(jnp.float32)
    p1 = jnp.where(take1, w, pmin)
    p0 = jnp.where(take1, pmin, w)
    zero = jnp.zeros((), jnp.float32)
    part = jnp.stack([
        jnp.sum(lse * lse),
        jnp.sum(p0),
        jnp.sum(p1),
        jnp.sum(t1f),
        zero, zero, zero, zero,
    ]).reshape(8, 1)

    @pl.when(i == 0)
    def _init():
        stats_ref[...] = jnp.zeros((8, 128), jnp.float32)

    stats_ref[:, 0:1] += part


@functools.partial(jax.jit, static_argnames=("block_t",))
def _run(hidden_states, Wr, Wb0, Wb1, block_t=1024):
    flat = hidden_states.reshape(N, D)
    wcat = jnp.stack(
        [Wb0.reshape(D, D // 128, 128), Wb1.reshape(D, D // 128, 128)],
        axis=2,
    ).reshape(D, 2 * D).astype(jnp.bfloat16)
    grid = N // block_t
    out, stats = pl.pallas_call(
        _fused_kernel,
        grid=(grid,),
        in_specs=[
            pl.BlockSpec((block_t, D), lambda i: (i, 0)),
            pl.BlockSpec((NREC, D), lambda i: (0, 0)),
            pl.BlockSpec((D, 2 * D), lambda i: (0, 0)),
        ],
        out_specs=[
            pl.BlockSpec((block_t, D), lambda i: (i, 0)),
            pl.BlockSpec((8, 128), lambda i: (0, 0)),
        ],
        out_shape=[
            jax.ShapeDtypeStruct((N, D), jnp.float32),
            jax.ShapeDtypeStruct((8, 128), jnp.float32),
        ],
        compiler_params=pltpu.CompilerParams(
            dimension_semantics=("arbitrary",),
        ),
    )(flat, Wr, wcat)

    lse2_sum = stats[0, 0]
    p0_sum = stats[1, 0]
    p1_sum = stats[2, 0]
    cnt1 = stats[3, 0]
    cnt0 = jnp.float32(N) - cnt1

    router_z_loss = lse2_sum / N
    expert_probs = jnp.stack([p0_sum, p1_sum]) / N
    expert_freq = jnp.stack([cnt0, cnt1]) / N
    balancing_loss = jnp.sum(expert_probs * expert_freq) * 0.1

    return out.reshape(B, S, D), router_z_loss, balancing_loss


def kernel(hidden_states, Wr, Wb0, Wb1):
    return _run(hidden_states, Wr, Wb0, Wb1)
